# baseline (speedup 1.0000x reference)
import functools

import jax
import jax.numpy as jnp
from jax.experimental import pallas as pl
from jax.experimental.pallas import tpu as pltpu

_EPS = 1e-12


def _prep_body(fw_ref, w1_ref, mn_ref, vp_ref, *, c, hdim):
    fw = fw_ref[...]
    m = fw[:, :c]
    n1 = jnp.sqrt(jnp.sum(m * m, axis=1, keepdims=True))
    mn_ref[...] = (m / jnp.maximum(n1, _EPS)).astype(jnp.bfloat16)
    n2 = jnp.sqrt(jnp.sum(fw * fw, axis=1, keepdims=True))
    fwn = fw / jnp.maximum(n2, _EPS)
    vp = jnp.dot(fwn, w1_ref[...], preferred_element_type=jnp.float32)
    bk = vp.shape[0]
    lane = jax.lax.broadcasted_iota(jnp.int32, (bk, 128), 1)
    ones_col = jnp.where(lane == 0, 1.0, 0.0).astype(jnp.float32)
    vp_ref[...] = jnp.concatenate([vp, ones_col], axis=1).astype(jnp.bfloat16)


def _main_body(xq_ref, mn_ref, vp_ref, b1_ref, w2_ref, b2_ref, out_ref):
    xq = xq_ref[...]
    nrm = jnp.sqrt(jnp.sum(xq * xq, axis=1, keepdims=True))
    xn = (xq * (1.4426950408889634 / jnp.maximum(nrm, _EPS))).astype(jnp.bfloat16)
    s = jax.lax.dot_general(
        xn, mn_ref[...], (((1,), (1,)), ((), ())),
        preferred_element_type=jnp.float32)
    e = jnp.exp2(s)
    hd = vp_ref.shape[1] - 128
    accd = jnp.dot(e.astype(jnp.bfloat16), vp_ref[...],
                   preferred_element_type=jnp.float32)
    acc = accd[:, :hd]
    den = accd[:, hd:hd + 1]
    o = acc / den + b1_ref[...]
    h1 = 0.5 * o * (1.0 + jax.lax.erf(o * (2.0 ** -0.5)))
    out_ref[...] = (jnp.dot(h1, w2_ref[...], preferred_element_type=jnp.float32)
                    + b2_ref[...])


def kernel(x, feat_w, W1, b1, W2, b2):
    b, c, h, w = x.shape
    n = b * h * w
    k, c4 = feat_w.shape
    hdim = W2.shape[1]

    x_flat = jnp.transpose(x, (0, 2, 3, 1)).reshape(n, c)
    b1_2d = b1.reshape(1, hdim)
    b2_2d = b2.reshape(1, hdim)

    bkp = 1024
    nkp = k // bkp
    vpw = hdim + 128
    mn, vp = pl.pallas_call(
        functools.partial(_prep_body, c=c, hdim=hdim),
        grid=(nkp,),
        in_specs=[
            pl.BlockSpec((bkp, c4), lambda i: (i, 0)),
            pl.BlockSpec((c4, hdim), lambda i: (0, 0)),
        ],
        out_specs=[
            pl.BlockSpec((bkp, c), lambda i: (i, 0)),
            pl.BlockSpec((bkp, vpw), lambda i: (i, 0)),
        ],
        out_shape=[
            jax.ShapeDtypeStruct((k, c), jnp.bfloat16),
            jax.ShapeDtypeStruct((k, vpw), jnp.bfloat16),
        ],
    )(feat_w, W1)

    bq = 1024
    nq = n // bq
    out2d = pl.pallas_call(
        _main_body,
        grid=(nq,),
        in_specs=[
            pl.BlockSpec((bq, c), lambda i: (i, 0)),
            pl.BlockSpec((k, c), lambda i: (0, 0)),
            pl.BlockSpec((k, vpw), lambda i: (0, 0)),
            pl.BlockSpec((1, hdim), lambda i: (0, 0)),
            pl.BlockSpec((hdim, hdim), lambda i: (0, 0)),
            pl.BlockSpec((1, hdim), lambda i: (0, 0)),
        ],
        out_specs=pl.BlockSpec((bq, hdim), lambda i: (i, 0)),
        out_shape=jax.ShapeDtypeStruct((n, hdim), jnp.float32),
        compiler_params=pltpu.CompilerParams(
            dimension_semantics=("arbitrary",)),
    )(x_flat, mn, vp, b1_2d, W2, b2_2d)

    return jnp.transpose(out2d.reshape(b, h, w, hdim), (0, 3, 1, 2))


# speedup vs baseline: 1.3224x; 1.3224x over previous
import functools

import jax
import jax.numpy as jnp
from jax.experimental import pallas as pl
from jax.experimental.pallas import tpu as pltpu

_EPS = 1e-12


def _prep_body(fw_ref, w1_ref, mn_ref, vp_ref, *, c, hdim):
    fw = fw_ref[...]
    m = fw[:, :c]
    n1 = jnp.sqrt(jnp.sum(m * m, axis=1, keepdims=True))
    mn_ref[...] = (m / jnp.maximum(n1, _EPS)).astype(jnp.bfloat16)
    n2 = jnp.sqrt(jnp.sum(fw * fw, axis=1, keepdims=True))
    fwn = fw / jnp.maximum(n2, _EPS)
    vp_ref[...] = jnp.dot(
        fwn, w1_ref[...], preferred_element_type=jnp.float32
    ).astype(jnp.bfloat16)


def _main_body(xq_ref, mn_ref, vp_ref, b1_ref, w2_ref, b2_ref, out_ref):
    xq = xq_ref[...]
    nrm = jnp.sqrt(jnp.sum(xq * xq, axis=1, keepdims=True))
    xn = (xq * (1.4426950408889634 / jnp.maximum(nrm, _EPS))).astype(jnp.bfloat16)
    s = jax.lax.dot_general(
        xn, mn_ref[...], (((1,), (1,)), ((), ())),
        preferred_element_type=jnp.float32)
    e = jnp.exp2(s)
    acc = jnp.dot(e.astype(jnp.bfloat16), vp_ref[...],
                  preferred_element_type=jnp.float32)
    den = jnp.sum(e, axis=1, keepdims=True)
    o = acc / den + b1_ref[...]
    h1 = 0.5 * o * (1.0 + jax.lax.erf(o * (2.0 ** -0.5)))
    out_ref[...] = (jnp.dot(h1, w2_ref[...], preferred_element_type=jnp.float32)
                    + b2_ref[...])


def kernel(x, feat_w, W1, b1, W2, b2):
    b, c, h, w = x.shape
    n = b * h * w
    k, c4 = feat_w.shape
    hdim = W2.shape[1]

    x_flat = jnp.transpose(x, (0, 2, 3, 1)).reshape(n, c)
    b1_2d = b1.reshape(1, hdim)
    b2_2d = b2.reshape(1, hdim)

    bkp = 1024
    nkp = k // bkp
    vpw = hdim
    mn, vp = pl.pallas_call(
        functools.partial(_prep_body, c=c, hdim=hdim),
        grid=(nkp,),
        in_specs=[
            pl.BlockSpec((bkp, c4), lambda i: (i, 0)),
            pl.BlockSpec((c4, hdim), lambda i: (0, 0)),
        ],
        out_specs=[
            pl.BlockSpec((bkp, c), lambda i: (i, 0)),
            pl.BlockSpec((bkp, vpw), lambda i: (i, 0)),
        ],
        out_shape=[
            jax.ShapeDtypeStruct((k, c), jnp.bfloat16),
            jax.ShapeDtypeStruct((k, vpw), jnp.bfloat16),
        ],
    )(feat_w, W1)

    bq = 1024
    nq = n // bq
    out2d = pl.pallas_call(
        _main_body,
        grid=(nq,),
        in_specs=[
            pl.BlockSpec((bq, c), lambda i: (i, 0)),
            pl.BlockSpec((k, c), lambda i: (0, 0)),
            pl.BlockSpec((k, vpw), lambda i: (0, 0)),
            pl.BlockSpec((1, hdim), lambda i: (0, 0)),
            pl.BlockSpec((hdim, hdim), lambda i: (0, 0)),
            pl.BlockSpec((1, hdim), lambda i: (0, 0)),
        ],
        out_specs=pl.BlockSpec((bq, hdim), lambda i: (i, 0)),
        out_shape=jax.ShapeDtypeStruct((n, hdim), jnp.float32),
        compiler_params=pltpu.CompilerParams(
            dimension_semantics=("arbitrary",)),
    )(x_flat, mn, vp, b1_2d, W2, b2_2d)

    return jnp.transpose(out2d.reshape(b, h, w, hdim), (0, 3, 1, 2))


# speedup vs baseline: 1.3235x; 1.0008x over previous
import functools

import jax
import jax.numpy as jnp
from jax.experimental import pallas as pl
from jax.experimental.pallas import tpu as pltpu

_EPS = 1e-12


def _prep_body(fw_ref, w1_ref, mn_ref, vp_ref, *, c, hdim):
    fw = fw_ref[...]
    m = fw[:, :c]
    n1 = jnp.sqrt(jnp.sum(m * m, axis=1, keepdims=True))
    mn_ref[...] = (m / jnp.maximum(n1, _EPS)).astype(jnp.bfloat16)
    n2 = jnp.sqrt(jnp.sum(fw * fw, axis=1, keepdims=True))
    fwn = fw / jnp.maximum(n2, _EPS)
    vp_ref[...] = jnp.dot(
        fwn, w1_ref[...], preferred_element_type=jnp.float32
    ).astype(jnp.bfloat16)


def _main_body(xq_ref, mn_ref, vp_ref, b1_ref, w2_ref, b2_ref, out_ref):
    xq = xq_ref[...]
    nrm = jnp.sqrt(jnp.sum(xq * xq, axis=1, keepdims=True))
    xn = (xq * (1.4426950408889634 / jnp.maximum(nrm, _EPS))).astype(jnp.bfloat16)
    s = jax.lax.dot_general(
        xn, mn_ref[...], (((1,), (1,)), ((), ())),
        preferred_element_type=jnp.float32)
    e = jnp.exp2(s.astype(jnp.bfloat16))
    acc = jnp.dot(e, vp_ref[...],
                  preferred_element_type=jnp.float32)
    kk = e.shape[1]
    e2 = e[:, :kk // 2] + e[:, kk // 2:]
    e4 = e2[:, :kk // 4] + e2[:, kk // 4:]
    den = jnp.sum(e4.astype(jnp.float32), axis=1, keepdims=True)
    o = acc / den + b1_ref[...]
    h1 = 0.5 * o * (1.0 + jax.lax.erf(o * (2.0 ** -0.5)))
    out_ref[...] = (jnp.dot(h1, w2_ref[...], preferred_element_type=jnp.float32)
                    + b2_ref[...])


def kernel(x, feat_w, W1, b1, W2, b2):
    b, c, h, w = x.shape
    n = b * h * w
    k, c4 = feat_w.shape
    hdim = W2.shape[1]

    x_flat = jnp.transpose(x, (0, 2, 3, 1)).reshape(n, c)
    b1_2d = b1.reshape(1, hdim)
    b2_2d = b2.reshape(1, hdim)

    bkp = 1024
    nkp = k // bkp
    vpw = hdim
    mn, vp = pl.pallas_call(
        functools.partial(_prep_body, c=c, hdim=hdim),
        grid=(nkp,),
        in_specs=[
            pl.BlockSpec((bkp, c4), lambda i: (i, 0)),
            pl.BlockSpec((c4, hdim), lambda i: (0, 0)),
        ],
        out_specs=[
            pl.BlockSpec((bkp, c), lambda i: (i, 0)),
            pl.BlockSpec((bkp, vpw), lambda i: (i, 0)),
        ],
        out_shape=[
            jax.ShapeDtypeStruct((k, c), jnp.bfloat16),
            jax.ShapeDtypeStruct((k, vpw), jnp.bfloat16),
        ],
    )(feat_w, W1)

    bq = 1024
    nq = n // bq
    out2d = pl.pallas_call(
        _main_body,
        grid=(nq,),
        in_specs=[
            pl.BlockSpec((bq, c), lambda i: (i, 0)),
            pl.BlockSpec((k, c), lambda i: (0, 0)),
            pl.BlockSpec((k, vpw), lambda i: (0, 0)),
            pl.BlockSpec((1, hdim), lambda i: (0, 0)),
            pl.BlockSpec((hdim, hdim), lambda i: (0, 0)),
            pl.BlockSpec((1, hdim), lambda i: (0, 0)),
        ],
        out_specs=pl.BlockSpec((bq, hdim), lambda i: (i, 0)),
        out_shape=jax.ShapeDtypeStruct((n, hdim), jnp.float32),
        compiler_params=pltpu.CompilerParams(
            dimension_semantics=("arbitrary",)),
    )(x_flat, mn, vp, b1_2d, W2, b2_2d)

    return jnp.transpose(out2d.reshape(b, h, w, hdim), (0, 3, 1, 2))


# speedup vs baseline: 1.4125x; 1.0673x over previous
import jax
import jax.numpy as jnp
from jax.experimental import pallas as pl
from jax.experimental.pallas import tpu as pltpu

_EPS = 1e-12
_LOG2E = 1.4426950408889634


def _body(xq_ref, fw_ref, w1_ref, b1_ref, w2_ref, b2_ref, out_ref,
          mn_ref, vp_ref):
    c = xq_ref.shape[1]

    @pl.when(pl.program_id(0) == 0)
    def _prep():
        fw = fw_ref[...]
        m = fw[:, :c]
        n1 = jnp.sqrt(jnp.sum(m * m, axis=1, keepdims=True))
        mn_ref[...] = (m / jnp.maximum(n1, _EPS)).astype(jnp.bfloat16)
        n2 = jnp.sqrt(jnp.sum(fw * fw, axis=1, keepdims=True))
        fwn = fw / jnp.maximum(n2, _EPS)
        vp_ref[...] = jnp.dot(
            fwn, w1_ref[...], preferred_element_type=jnp.float32
        ).astype(jnp.bfloat16)

    xq = xq_ref[...]
    nrm = jnp.sqrt(jnp.sum(xq * xq, axis=1, keepdims=True))
    xn = (xq * (_LOG2E / jnp.maximum(nrm, _EPS))).astype(jnp.bfloat16)
    s = jax.lax.dot_general(
        xn, mn_ref[...], (((1,), (1,)), ((), ())),
        preferred_element_type=jnp.float32)
    e = jnp.exp2(s.astype(jnp.bfloat16))
    acc = jnp.dot(e, vp_ref[...],
                  preferred_element_type=jnp.float32)
    kk = e.shape[1]
    e2 = e[:, :kk // 2] + e[:, kk // 2:]
    e4 = e2[:, :kk // 4] + e2[:, kk // 4:]
    den = jnp.sum(e4.astype(jnp.float32), axis=1, keepdims=True)
    o = acc / den + b1_ref[...]
    h1 = 0.5 * o * (1.0 + jax.lax.erf(o * (2.0 ** -0.5)))
    out_ref[...] = (jnp.dot(h1, w2_ref[...], preferred_element_type=jnp.float32)
                    + b2_ref[...])


def kernel(x, feat_w, W1, b1, W2, b2):
    b, c, h, w = x.shape
    n = b * h * w
    k, c4 = feat_w.shape
    hdim = W2.shape[1]

    x_flat = jnp.transpose(x, (0, 2, 3, 1)).reshape(n, c)
    b1_2d = b1.reshape(1, hdim)
    b2_2d = b2.reshape(1, hdim)

    bq = 1024
    nq = n // bq
    out2d = pl.pallas_call(
        _body,
        grid=(nq,),
        in_specs=[
            pl.BlockSpec((bq, c), lambda i: (i, 0)),
            pl.BlockSpec((k, c4), lambda i: (0, 0)),
            pl.BlockSpec((c4, hdim), lambda i: (0, 0)),
            pl.BlockSpec((1, hdim), lambda i: (0, 0)),
            pl.BlockSpec((hdim, hdim), lambda i: (0, 0)),
            pl.BlockSpec((1, hdim), lambda i: (0, 0)),
        ],
        out_specs=pl.BlockSpec((bq, hdim), lambda i: (i, 0)),
        out_shape=jax.ShapeDtypeStruct((n, hdim), jnp.float32),
        scratch_shapes=[
            pltpu.VMEM((k, c), jnp.bfloat16),
            pltpu.VMEM((k, hdim), jnp.bfloat16),
        ],
        compiler_params=pltpu.CompilerParams(
            dimension_semantics=("arbitrary",)),
    )(x_flat, feat_w, W1, b1_2d, W2, b2_2d)

    return jnp.transpose(out2d.reshape(b, h, w, hdim), (0, 3, 1, 2))
